# Initial kernel scaffold; baseline (speedup 1.0000x reference)
import jax
import jax.numpy as jnp
from jax.experimental import pallas as pl


def kernel(pos_h, pos_t, pos_r, neg_h, neg_t, neg_r, ent_emb, rel_emb, wrs_emb, wro_emb, brs_emb, bro_emb):
    raise NotImplementedError("write your pallas kernel here")


# speedup vs baseline: 1.3430x; 1.3430x over previous
import functools

import jax
import jax.numpy as jnp
from jax import lax
from jax.experimental import pallas as pl
from jax.experimental.pallas import tpu as pltpu
from jax.experimental.pallas import tpu_sc as plsc

_ENT = 1000000
_REL = 1000
_D = 64
_B = 16384
_MARGIN = 1.0
_LS = 0.3
_LO = 0.3

_NW = 32
_PW = _B // _NW
_C = 64
_NCH = _PW // _C
_PK = 5 * _D


def _l2n_tc(x):
    return x * lax.rsqrt(jnp.maximum(jnp.sum(x * x, axis=-1, keepdims=True), 1e-12))


def _prep_body(rel_ref, wrs_ref, wro_ref, brs_ref, bro_ref, out_ref):
    rel = rel_ref[...]
    wrs = wrs_ref[...]
    wro = wro_ref[...]
    rn = _l2n_tc(rel)
    rh = _l2n_tc(brs_ref[...] * wrs)
    rt = _l2n_tc(bro_ref[...] * wro)
    out_ref[...] = jnp.concatenate([rn, wrs, wro, rh, rt], axis=1)


def _prep_rel(rel_emb, wrs_emb, wro_emb, brs_emb, bro_emb):
    return pl.pallas_call(
        _prep_body,
        out_shape=jax.ShapeDtypeStruct((_REL, _PK), jnp.float32),
    )(rel_emb, wrs_emb, wro_emb, brs_emb, bro_emb)


def _rsqrt16(x):
    x = jnp.maximum(x, 1e-12)
    i = lax.bitcast_convert_type(x, jnp.int32)
    i = jnp.int32(0x5F3759DF) - (i >> 1)
    y = lax.bitcast_convert_type(i, jnp.float32)
    for _ in range(3):
        y = y * (1.5 - 0.5 * x * y * y)
    return y


def _sc_kernel_fn(ent_hbm, rpk_hbm, ph_hbm, pt_hbm, pr_hbm, nh_hbm, nt_hbm, nr_hbm,
                  out_hbm,
                  ph_v, pt_v, pr_v, nh_v, nt_v, nr_v,
                  hp_v, tp_v, hn_v, tn_v, rp_v, rn_v, acc_v, sem):
    info = plsc.get_sparse_core_info()
    nc = info.num_cores
    wid = lax.axis_index("s") * nc + lax.axis_index("c")
    base = wid * _PW

    pltpu.sync_copy(ph_hbm.at[pl.ds(base, _PW)], ph_v)
    pltpu.sync_copy(pt_hbm.at[pl.ds(base, _PW)], pt_v)
    pltpu.sync_copy(pr_hbm.at[pl.ds(base, _PW)], pr_v)
    pltpu.sync_copy(nh_hbm.at[pl.ds(base, _PW)], nh_v)
    pltpu.sync_copy(nt_hbm.at[pl.ds(base, _PW)], nt_v)
    pltpu.sync_copy(nr_hbm.at[pl.ds(base, _PW)], nr_v)

    lanes = lax.iota(jnp.int32, 16)
    lane0 = lanes == 0

    dnums = lax.GatherDimensionNumbers(
        offset_dims=(), collapsed_slice_dims=(0,), start_index_map=(0,))

    def shuf(v, idx):
        return lax.gather(v, idx[:, None], dnums, (1,),
                          mode=lax.GatherScatterMode.PROMISE_IN_BOUNDS)

    def bsum(v):
        for sh in (1, 2, 4, 8):
            v = v + shuf(v, lanes ^ sh)
        return v

    def bcast_lane(v, k):
        return shuf(v, jnp.full((16,), k, jnp.int32))

    acc = jnp.zeros((16,), jnp.float32)
    for ch in range(_NCH):
        off = ch * _C
        d1 = pltpu.async_copy(ent_hbm.at[ph_v.at[pl.ds(off, _C)]], hp_v, sem)
        d2 = pltpu.async_copy(ent_hbm.at[pt_v.at[pl.ds(off, _C)]], tp_v, sem)
        d3 = pltpu.async_copy(ent_hbm.at[nh_v.at[pl.ds(off, _C)]], hn_v, sem)
        d4 = pltpu.async_copy(ent_hbm.at[nt_v.at[pl.ds(off, _C)]], tn_v, sem)
        d5 = pltpu.async_copy(rpk_hbm.at[pr_v.at[pl.ds(off, _C)]], rp_v, sem)
        d6 = pltpu.async_copy(rpk_hbm.at[nr_v.at[pl.ds(off, _C)]], rn_v, sem)
        d1.wait(); d2.wait(); d3.wait(); d4.wait(); d5.wait(); d6.wait()

        def body(e, acc):
            hp = [hp_v[e, pl.ds(16 * k, 16)] for k in range(4)]
            tp = [tp_v[e, pl.ds(16 * k, 16)] for k in range(4)]
            hn = [hn_v[e, pl.ds(16 * k, 16)] for k in range(4)]
            tn = [tn_v[e, pl.ds(16 * k, 16)] for k in range(4)]
            wsp = [rp_v[e, pl.ds(_D + 16 * k, 16)] for k in range(4)]
            wop = [rp_v[e, pl.ds(2 * _D + 16 * k, 16)] for k in range(4)]
            wsn = [rn_v[e, pl.ds(_D + 16 * k, 16)] for k in range(4)]
            won = [rn_v[e, pl.ds(2 * _D + 16 * k, 16)] for k in range(4)]

            hwp = [hp[k] * wsp[k] for k in range(4)]
            twp = [tp[k] * wop[k] for k in range(4)]
            hwn = [hn[k] * wsn[k] for k in range(4)]
            twn = [tn[k] * won[k] for k in range(4)]

            def ssq(vs):
                a = vs[0] * vs[0]
                for k in range(1, 4):
                    a = a + vs[k] * vs[k]
                return bsum(a)

            sums = [ssq(hp), ssq(tp), ssq(hwp), ssq(twp),
                    ssq(hn), ssq(tn), ssq(hwn), ssq(twn)]
            x = sums[0]
            for k in range(1, 8):
                x = jnp.where(lanes == k, sums[k], x)
            y = _rsqrt16(x)
            ihp, itp, ihwp, itwp, ihn, itn, ihwn, itwn = [
                bcast_lane(y, k) for k in range(8)]

            s1 = jnp.zeros((16,), jnp.float32)
            s2 = jnp.zeros((16,), jnp.float32)
            for k in range(4):
                rnp = rp_v[e, pl.ds(16 * k, 16)]
                rhp = rp_v[e, pl.ds(3 * _D + 16 * k, 16)]
                rtp = rp_v[e, pl.ds(4 * _D + 16 * k, 16)]
                rnn = rn_v[e, pl.ds(16 * k, 16)]
                rhn = rn_v[e, pl.ds(3 * _D + 16 * k, 16)]
                rtn = rn_v[e, pl.ds(4 * _D + 16 * k, 16)]
                ap = jnp.abs(hp[k] * ihp + rnp - tp[k] * itp)
                an = jnp.abs(hn[k] * ihn + rnn - tn[k] * itn)
                bp = jnp.abs(hwp[k] * ihwp - rhp) + jnp.abs(twp[k] * itwp - rtp)
                bn = jnp.abs(hwn[k] * ihwn - rhn) + jnp.abs(twn[k] * itwn - rtn)
                s1 = s1 + (ap - an)
                s2 = s2 + (bp - bn)
            dv = bsum(s1 + _LS * s2)
            hinge = jnp.maximum(dv + _MARGIN, 0.0)
            return acc + jnp.where(lane0, hinge, 0.0)

        acc = lax.fori_loop(0, _C, body, acc)

    acc_v[...] = acc
    pltpu.sync_copy(acc_v, out_hbm.at[wid])


def _sc_call(ent_emb, rel_packed, ph, pt, pr, nh, nt, nr):
    mesh = plsc.VectorSubcoreMesh(core_axis_name="c", subcore_axis_name="s")
    f = functools.partial(
        pl.kernel,
        mesh=mesh,
        compiler_params=pltpu.CompilerParams(use_tc_tiling_on_sc=False),
        out_type=jax.ShapeDtypeStruct((_NW, 16), jnp.float32),
        scratch_types=[
            pltpu.VMEM((_PW,), jnp.int32),
            pltpu.VMEM((_PW,), jnp.int32),
            pltpu.VMEM((_PW,), jnp.int32),
            pltpu.VMEM((_PW,), jnp.int32),
            pltpu.VMEM((_PW,), jnp.int32),
            pltpu.VMEM((_PW,), jnp.int32),
            pltpu.VMEM((_C, _D), jnp.float32),
            pltpu.VMEM((_C, _D), jnp.float32),
            pltpu.VMEM((_C, _D), jnp.float32),
            pltpu.VMEM((_C, _D), jnp.float32),
            pltpu.VMEM((_C, _PK), jnp.float32),
            pltpu.VMEM((_C, _PK), jnp.float32),
            pltpu.VMEM((16,), jnp.float32),
            pltpu.SemaphoreType.DMA,
        ],
    )(_sc_kernel_fn)
    return f(ent_emb, rel_packed, ph, pt, pr, nh, nt, nr)


def kernel(pos_h, pos_t, pos_r, neg_h, neg_t, neg_r,
           ent_emb, rel_emb, wrs_emb, wro_emb, brs_emb, bro_emb):
    rel_packed = _prep_rel(rel_emb, wrs_emb, wro_emb, brs_emb, bro_emb)
    parts = _sc_call(
        ent_emb, rel_packed,
        pos_h.astype(jnp.int32), pos_t.astype(jnp.int32), pos_r.astype(jnp.int32),
        neg_h.astype(jnp.int32), neg_t.astype(jnp.int32), neg_r.astype(jnp.int32),
    )
    return jnp.sum(parts)


# speedup vs baseline: 1.3592x; 1.0121x over previous
import functools

import jax
import jax.numpy as jnp
from jax import lax
from jax.experimental import pallas as pl
from jax.experimental.pallas import tpu as pltpu
from jax.experimental.pallas import tpu_sc as plsc

_ENT = 1000000
_REL = 1000
_D = 64
_B = 16384
_MARGIN = 1.0
_LS = 0.3
_LO = 0.3

_NW = 32
_PW = _B // _NW
_C = 64
_NCH = _PW // _C
_PK = 5 * _D


def _l2n_tc(x):
    return x * lax.rsqrt(jnp.maximum(jnp.sum(x * x, axis=-1, keepdims=True), 1e-12))


def _prep_body(rel_ref, wrs_ref, wro_ref, brs_ref, bro_ref, out_ref):
    rel = rel_ref[...]
    wrs = wrs_ref[...]
    wro = wro_ref[...]
    rn = _l2n_tc(rel)
    rh = _l2n_tc(brs_ref[...] * wrs)
    rt = _l2n_tc(bro_ref[...] * wro)
    out_ref[...] = jnp.concatenate([rn, wrs, wro, rh, rt], axis=1)


def _prep_rel(rel_emb, wrs_emb, wro_emb, brs_emb, bro_emb):
    return pl.pallas_call(
        _prep_body,
        out_shape=jax.ShapeDtypeStruct((_REL, _PK), jnp.float32),
    )(rel_emb, wrs_emb, wro_emb, brs_emb, bro_emb)


def _rsqrt16(x):
    x = jnp.maximum(x, 1e-12)
    i = lax.bitcast_convert_type(x, jnp.int32)
    i = jnp.int32(0x5F3759DF) - (i >> 1)
    y = lax.bitcast_convert_type(i, jnp.float32)
    for _ in range(3):
        y = y * (1.5 - 0.5 * x * y * y)
    return y


def _sc_kernel_fn(ent_hbm, rpk_hbm, ph_hbm, pt_hbm, pr_hbm, nh_hbm, nt_hbm, nr_hbm,
                  out_hbm,
                  ph_v, pt_v, pr_v, nh_v, nt_v, nr_v,
                  pht_v, ptt_v, nht_v, ntt_v,
                  hp_v, tp_v, hn_v, tn_v, rp_v, rn_v, acc_v, sem):
    info = plsc.get_sparse_core_info()
    nc = info.num_cores
    wid = lax.axis_index("s") * nc + lax.axis_index("c")
    base = wid * _PW

    pltpu.sync_copy(ph_hbm.at[pl.ds(base, _PW)], ph_v)
    pltpu.sync_copy(pt_hbm.at[pl.ds(base, _PW)], pt_v)
    pltpu.sync_copy(pr_hbm.at[pl.ds(base, _PW)], pr_v)
    pltpu.sync_copy(nh_hbm.at[pl.ds(base, _PW)], nh_v)
    pltpu.sync_copy(nt_hbm.at[pl.ds(base, _PW)], nt_v)
    pltpu.sync_copy(nr_hbm.at[pl.ds(base, _PW)], nr_v)

    def half(i, _):
        s = pl.ds(i * 16, 16)
        pht_v[s] = ph_v[s] >> 1
        ptt_v[s] = pt_v[s] >> 1
        nht_v[s] = nh_v[s] >> 1
        ntt_v[s] = nt_v[s] >> 1
        ph_v[s] = ph_v[s] & 1
        pt_v[s] = pt_v[s] & 1
        nh_v[s] = nh_v[s] & 1
        nt_v[s] = nt_v[s] & 1
        return 0
    lax.fori_loop(0, _PW // 16, half, 0)

    lanes = lax.iota(jnp.int32, 16)
    lane0 = lanes == 0

    dnums = lax.GatherDimensionNumbers(
        offset_dims=(), collapsed_slice_dims=(0,), start_index_map=(0,))

    def shuf(v, idx):
        return lax.gather(v, idx[:, None], dnums, (1,),
                          mode=lax.GatherScatterMode.PROMISE_IN_BOUNDS)

    def bsum(v):
        for sh in (1, 2, 4, 8):
            v = v + shuf(v, lanes ^ sh)
        return v

    def bcast_lane(v, k):
        return shuf(v, jnp.full((16,), k, jnp.int32))

    acc = jnp.zeros((16,), jnp.float32)
    for ch in range(_NCH):
        off = ch * _C
        d1 = pltpu.async_copy(ent_hbm.at[pht_v.at[pl.ds(off, _C)]], hp_v, sem)
        d2 = pltpu.async_copy(ent_hbm.at[ptt_v.at[pl.ds(off, _C)]], tp_v, sem)
        d3 = pltpu.async_copy(ent_hbm.at[nht_v.at[pl.ds(off, _C)]], hn_v, sem)
        d4 = pltpu.async_copy(ent_hbm.at[ntt_v.at[pl.ds(off, _C)]], tn_v, sem)
        d5 = pltpu.async_copy(rpk_hbm.at[pr_v.at[pl.ds(off, _C)]], rp_v, sem)
        d6 = pltpu.async_copy(rpk_hbm.at[nr_v.at[pl.ds(off, _C)]], rn_v, sem)
        d1.wait(); d2.wait(); d3.wait(); d4.wait(); d5.wait(); d6.wait()

        def body(e, acc):
            ebase = off + e
            al = ebase & -16
            lanev = jnp.full((16,), ebase & 15, jnp.int32)

            def parb(buf):
                return shuf(buf[pl.ds(al, 16)], lanev).astype(jnp.float32)

            php = parb(ph_v)
            ptp = parb(pt_v)
            nhp = parb(nh_v)
            ntp = parb(nt_v)

            def sel(buf, par, k):
                lo = buf[e, pl.ds(16 * k, 16)]
                hi = buf[e, pl.ds(_D + 16 * k, 16)]
                return lo * (1.0 - par) + hi * par

            hp = [sel(hp_v, php, k) for k in range(4)]
            tp = [sel(tp_v, ptp, k) for k in range(4)]
            hn = [sel(hn_v, nhp, k) for k in range(4)]
            tn = [sel(tn_v, ntp, k) for k in range(4)]
            wsp = [rp_v[e, pl.ds(_D + 16 * k, 16)] for k in range(4)]
            wop = [rp_v[e, pl.ds(2 * _D + 16 * k, 16)] for k in range(4)]
            wsn = [rn_v[e, pl.ds(_D + 16 * k, 16)] for k in range(4)]
            won = [rn_v[e, pl.ds(2 * _D + 16 * k, 16)] for k in range(4)]

            hwp = [hp[k] * wsp[k] for k in range(4)]
            twp = [tp[k] * wop[k] for k in range(4)]
            hwn = [hn[k] * wsn[k] for k in range(4)]
            twn = [tn[k] * won[k] for k in range(4)]

            def ssq(vs):
                a = vs[0] * vs[0]
                for k in range(1, 4):
                    a = a + vs[k] * vs[k]
                return bsum(a)

            sums = [ssq(hp), ssq(tp), ssq(hwp), ssq(twp),
                    ssq(hn), ssq(tn), ssq(hwn), ssq(twn)]
            x = sums[0]
            for k in range(1, 8):
                x = jnp.where(lanes == k, sums[k], x)
            y = _rsqrt16(x)
            ihp, itp, ihwp, itwp, ihn, itn, ihwn, itwn = [
                bcast_lane(y, k) for k in range(8)]

            s1 = jnp.zeros((16,), jnp.float32)
            s2 = jnp.zeros((16,), jnp.float32)
            for k in range(4):
                rnp = rp_v[e, pl.ds(16 * k, 16)]
                rhp = rp_v[e, pl.ds(3 * _D + 16 * k, 16)]
                rtp = rp_v[e, pl.ds(4 * _D + 16 * k, 16)]
                rnn = rn_v[e, pl.ds(16 * k, 16)]
                rhn = rn_v[e, pl.ds(3 * _D + 16 * k, 16)]
                rtn = rn_v[e, pl.ds(4 * _D + 16 * k, 16)]
                ap = jnp.abs(hp[k] * ihp + rnp - tp[k] * itp)
                an = jnp.abs(hn[k] * ihn + rnn - tn[k] * itn)
                bp = jnp.abs(hwp[k] * ihwp - rhp) + jnp.abs(twp[k] * itwp - rtp)
                bn = jnp.abs(hwn[k] * ihwn - rhn) + jnp.abs(twn[k] * itwn - rtn)
                s1 = s1 + (ap - an)
                s2 = s2 + (bp - bn)
            dv = bsum(s1 + _LS * s2)
            hinge = jnp.maximum(dv + _MARGIN, 0.0)
            return acc + jnp.where(lane0, hinge, 0.0)

        acc = lax.fori_loop(0, _C, body, acc)

    acc_v[...] = acc
    pltpu.sync_copy(acc_v, out_hbm.at[wid])


def _sc_call(ent_emb, rel_packed, ph, pt, pr, nh, nt, nr):
    mesh = plsc.VectorSubcoreMesh(core_axis_name="c", subcore_axis_name="s")
    f = functools.partial(
        pl.kernel,
        mesh=mesh,
        compiler_params=pltpu.CompilerParams(use_tc_tiling_on_sc=False),
        out_type=jax.ShapeDtypeStruct((_NW, 16), jnp.float32),
        scratch_types=[
            pltpu.VMEM((_PW,), jnp.int32),
            pltpu.VMEM((_PW,), jnp.int32),
            pltpu.VMEM((_PW,), jnp.int32),
            pltpu.VMEM((_PW,), jnp.int32),
            pltpu.VMEM((_PW,), jnp.int32),
            pltpu.VMEM((_PW,), jnp.int32),
            pltpu.VMEM((_PW,), jnp.int32),
            pltpu.VMEM((_PW,), jnp.int32),
            pltpu.VMEM((_PW,), jnp.int32),
            pltpu.VMEM((_PW,), jnp.int32),
            pltpu.VMEM((_C, 2 * _D), jnp.float32),
            pltpu.VMEM((_C, 2 * _D), jnp.float32),
            pltpu.VMEM((_C, 2 * _D), jnp.float32),
            pltpu.VMEM((_C, 2 * _D), jnp.float32),
            pltpu.VMEM((_C, _PK), jnp.float32),
            pltpu.VMEM((_C, _PK), jnp.float32),
            pltpu.VMEM((16,), jnp.float32),
            pltpu.SemaphoreType.DMA,
        ],
    )(_sc_kernel_fn)
    return f(ent_emb, rel_packed, ph, pt, pr, nh, nt, nr)


def kernel(pos_h, pos_t, pos_r, neg_h, neg_t, neg_r,
           ent_emb, rel_emb, wrs_emb, wro_emb, brs_emb, bro_emb):
    rel_packed = _prep_rel(rel_emb, wrs_emb, wro_emb, brs_emb, bro_emb)
    parts = _sc_call(
        ent_emb.reshape(_ENT // 2, 2 * _D), rel_packed,
        pos_h.astype(jnp.int32), pos_t.astype(jnp.int32), pos_r.astype(jnp.int32),
        neg_h.astype(jnp.int32), neg_t.astype(jnp.int32), neg_r.astype(jnp.int32),
    )
    return jnp.sum(parts)


# speedup vs baseline: 2.7932x; 2.0550x over previous
import functools

import jax
import jax.numpy as jnp
from jax import lax
from jax.experimental import pallas as pl
from jax.experimental.pallas import tpu as pltpu
from jax.experimental.pallas import tpu_sc as plsc

_ENT = 1000000
_REL = 1000
_D = 64
_B = 16384
_MARGIN = 1.0
_LS = 0.3
_LO = 0.3

_NW = 32
_PW = _B // _NW
_C = 64
_NCH = _PW // _C
_PK = 5 * _D


def _l2n_tc(x):
    return x * lax.rsqrt(jnp.maximum(jnp.sum(x * x, axis=-1, keepdims=True), 1e-12))


def _prep_body(rel_ref, wrs_ref, wro_ref, brs_ref, bro_ref, out_ref):
    rel = rel_ref[...]
    wrs = wrs_ref[...]
    wro = wro_ref[...]
    rn = _l2n_tc(rel)
    rh = _l2n_tc(brs_ref[...] * wrs)
    rt = _l2n_tc(bro_ref[...] * wro)
    out_ref[...] = jnp.concatenate([rn, wrs, wro, rh, rt], axis=1)


def _prep_rel(rel_emb, wrs_emb, wro_emb, brs_emb, bro_emb):
    return pl.pallas_call(
        _prep_body,
        out_shape=jax.ShapeDtypeStruct((_REL, _PK), jnp.float32),
    )(rel_emb, wrs_emb, wro_emb, brs_emb, bro_emb)


_BN = 8192
_NBLK = 62
_S = _NBLK * _BN


def _relay_body(top_ref, bot_ref, out_ref):
    out_ref[:, 0:_D] = top_ref[...].T
    out_ref[:, _D:2 * _D] = bot_ref[...].T


def _relayout(ent_t):
    return pl.pallas_call(
        _relay_body,
        grid=(_NBLK,),
        in_specs=[
            pl.BlockSpec((_D, _BN), lambda j: (0, j)),
            pl.BlockSpec((_D, _BN),
                         lambda j: (0, jnp.minimum(_NBLK + j, _ENT // _BN))),
        ],
        out_specs=pl.BlockSpec((_BN, 2 * _D), lambda j: (j, 0)),
        out_shape=jax.ShapeDtypeStruct((_S, 2 * _D), jnp.float32),
    )(ent_t, ent_t)


def _rsqrt16(x):
    x = jnp.maximum(x, 1e-12)
    i = lax.bitcast_convert_type(x, jnp.int32)
    i = jnp.int32(0x5F3759DF) - (i >> 1)
    y = lax.bitcast_convert_type(i, jnp.float32)
    for _ in range(3):
        y = y * (1.5 - 0.5 * x * y * y)
    return y


def _sc_kernel_fn(ent_hbm, rpk_hbm, ph_hbm, pt_hbm, pr_hbm, nh_hbm, nt_hbm, nr_hbm,
                  out_hbm,
                  ph_v, pt_v, pr_v, nh_v, nt_v, nr_v,
                  pht_v, ptt_v, nht_v, ntt_v,
                  hp_v, tp_v, hn_v, tn_v, rp_v, rn_v, acc_v, sem):
    info = plsc.get_sparse_core_info()
    nc = info.num_cores
    wid = lax.axis_index("s") * nc + lax.axis_index("c")
    base = wid * _PW

    pltpu.sync_copy(ph_hbm.at[pl.ds(base, _PW)], ph_v)
    pltpu.sync_copy(pt_hbm.at[pl.ds(base, _PW)], pt_v)
    pltpu.sync_copy(pr_hbm.at[pl.ds(base, _PW)], pr_v)
    pltpu.sync_copy(nh_hbm.at[pl.ds(base, _PW)], nh_v)
    pltpu.sync_copy(nt_hbm.at[pl.ds(base, _PW)], nt_v)
    pltpu.sync_copy(nr_hbm.at[pl.ds(base, _PW)], nr_v)

    def half(i, _):
        s = pl.ds(i * 16, 16)
        for idx_b, row_b in ((ph_v, pht_v), (pt_v, ptt_v),
                             (nh_v, nht_v), (nt_v, ntt_v)):
            v = idx_b[s]
            d = v - _S
            g = 1 - ((d >> 31) & 1)
            row_b[s] = v - g * _S
            idx_b[s] = g
        return 0
    lax.fori_loop(0, _PW // 16, half, 0)

    lanes = lax.iota(jnp.int32, 16)
    lane0 = lanes == 0

    dnums = lax.GatherDimensionNumbers(
        offset_dims=(), collapsed_slice_dims=(0,), start_index_map=(0,))

    def shuf(v, idx):
        return lax.gather(v, idx[:, None], dnums, (1,),
                          mode=lax.GatherScatterMode.PROMISE_IN_BOUNDS)

    def bsum(v):
        for sh in (1, 2, 4, 8):
            v = v + shuf(v, lanes ^ sh)
        return v

    def bcast_lane(v, k):
        return shuf(v, jnp.full((16,), k, jnp.int32))

    acc = jnp.zeros((16,), jnp.float32)
    for ch in range(_NCH):
        off = ch * _C
        d1 = pltpu.async_copy(ent_hbm.at[pht_v.at[pl.ds(off, _C)]], hp_v, sem)
        d2 = pltpu.async_copy(ent_hbm.at[ptt_v.at[pl.ds(off, _C)]], tp_v, sem)
        d3 = pltpu.async_copy(ent_hbm.at[nht_v.at[pl.ds(off, _C)]], hn_v, sem)
        d4 = pltpu.async_copy(ent_hbm.at[ntt_v.at[pl.ds(off, _C)]], tn_v, sem)
        d5 = pltpu.async_copy(rpk_hbm.at[pr_v.at[pl.ds(off, _C)]], rp_v, sem)
        d6 = pltpu.async_copy(rpk_hbm.at[nr_v.at[pl.ds(off, _C)]], rn_v, sem)
        d1.wait(); d2.wait(); d3.wait(); d4.wait(); d5.wait(); d6.wait()

        def body(e, acc):
            ebase = off + e
            al = ebase & -16
            lanev = jnp.full((16,), ebase & 15, jnp.int32)

            def parb(buf):
                return shuf(buf[pl.ds(al, 16)], lanev).astype(jnp.float32)

            php = parb(ph_v)
            ptp = parb(pt_v)
            nhp = parb(nh_v)
            ntp = parb(nt_v)

            def sel(buf, par, k):
                lo = buf[e, pl.ds(16 * k, 16)]
                hi = buf[e, pl.ds(_D + 16 * k, 16)]
                return lo * (1.0 - par) + hi * par

            hp = [sel(hp_v, php, k) for k in range(4)]
            tp = [sel(tp_v, ptp, k) for k in range(4)]
            hn = [sel(hn_v, nhp, k) for k in range(4)]
            tn = [sel(tn_v, ntp, k) for k in range(4)]
            wsp = [rp_v[e, pl.ds(_D + 16 * k, 16)] for k in range(4)]
            wop = [rp_v[e, pl.ds(2 * _D + 16 * k, 16)] for k in range(4)]
            wsn = [rn_v[e, pl.ds(_D + 16 * k, 16)] for k in range(4)]
            won = [rn_v[e, pl.ds(2 * _D + 16 * k, 16)] for k in range(4)]

            hwp = [hp[k] * wsp[k] for k in range(4)]
            twp = [tp[k] * wop[k] for k in range(4)]
            hwn = [hn[k] * wsn[k] for k in range(4)]
            twn = [tn[k] * won[k] for k in range(4)]

            def ssq(vs):
                a = vs[0] * vs[0]
                for k in range(1, 4):
                    a = a + vs[k] * vs[k]
                return bsum(a)

            sums = [ssq(hp), ssq(tp), ssq(hwp), ssq(twp),
                    ssq(hn), ssq(tn), ssq(hwn), ssq(twn)]
            x = sums[0]
            for k in range(1, 8):
                x = jnp.where(lanes == k, sums[k], x)
            y = _rsqrt16(x)
            ihp, itp, ihwp, itwp, ihn, itn, ihwn, itwn = [
                bcast_lane(y, k) for k in range(8)]

            s1 = jnp.zeros((16,), jnp.float32)
            s2 = jnp.zeros((16,), jnp.float32)
            for k in range(4):
                rnp = rp_v[e, pl.ds(16 * k, 16)]
                rhp = rp_v[e, pl.ds(3 * _D + 16 * k, 16)]
                rtp = rp_v[e, pl.ds(4 * _D + 16 * k, 16)]
                rnn = rn_v[e, pl.ds(16 * k, 16)]
                rhn = rn_v[e, pl.ds(3 * _D + 16 * k, 16)]
                rtn = rn_v[e, pl.ds(4 * _D + 16 * k, 16)]
                ap = jnp.abs(hp[k] * ihp + rnp - tp[k] * itp)
                an = jnp.abs(hn[k] * ihn + rnn - tn[k] * itn)
                bp = jnp.abs(hwp[k] * ihwp - rhp) + jnp.abs(twp[k] * itwp - rtp)
                bn = jnp.abs(hwn[k] * ihwn - rhn) + jnp.abs(twn[k] * itwn - rtn)
                s1 = s1 + (ap - an)
                s2 = s2 + (bp - bn)
            dv = bsum(s1 + _LS * s2)
            hinge = jnp.maximum(dv + _MARGIN, 0.0)
            return acc + jnp.where(lane0, hinge, 0.0)

        acc = lax.fori_loop(0, _C, body, acc)

    acc_v[...] = acc
    pltpu.sync_copy(acc_v, out_hbm.at[wid])


def _sc_call(ent_emb, rel_packed, ph, pt, pr, nh, nt, nr):
    mesh = plsc.VectorSubcoreMesh(core_axis_name="c", subcore_axis_name="s")
    f = functools.partial(
        pl.kernel,
        mesh=mesh,
        compiler_params=pltpu.CompilerParams(use_tc_tiling_on_sc=False),
        out_type=jax.ShapeDtypeStruct((_NW, 16), jnp.float32),
        scratch_types=[
            pltpu.VMEM((_PW,), jnp.int32),
            pltpu.VMEM((_PW,), jnp.int32),
            pltpu.VMEM((_PW,), jnp.int32),
            pltpu.VMEM((_PW,), jnp.int32),
            pltpu.VMEM((_PW,), jnp.int32),
            pltpu.VMEM((_PW,), jnp.int32),
            pltpu.VMEM((_PW,), jnp.int32),
            pltpu.VMEM((_PW,), jnp.int32),
            pltpu.VMEM((_PW,), jnp.int32),
            pltpu.VMEM((_PW,), jnp.int32),
            pltpu.VMEM((_C, 2 * _D), jnp.float32),
            pltpu.VMEM((_C, 2 * _D), jnp.float32),
            pltpu.VMEM((_C, 2 * _D), jnp.float32),
            pltpu.VMEM((_C, 2 * _D), jnp.float32),
            pltpu.VMEM((_C, _PK), jnp.float32),
            pltpu.VMEM((_C, _PK), jnp.float32),
            pltpu.VMEM((16,), jnp.float32),
            pltpu.SemaphoreType.DMA,
        ],
    )(_sc_kernel_fn)
    return f(ent_emb, rel_packed, ph, pt, pr, nh, nt, nr)


def kernel(pos_h, pos_t, pos_r, neg_h, neg_t, neg_r,
           ent_emb, rel_emb, wrs_emb, wro_emb, brs_emb, bro_emb):
    rel_packed = _prep_rel(rel_emb, wrs_emb, wro_emb, brs_emb, bro_emb)
    ent_pairs = _relayout(ent_emb.T)
    parts = _sc_call(
        ent_pairs, rel_packed,
        pos_h.astype(jnp.int32), pos_t.astype(jnp.int32), pos_r.astype(jnp.int32),
        neg_h.astype(jnp.int32), neg_t.astype(jnp.int32), neg_r.astype(jnp.int32),
    )
    return jnp.sum(parts)


# speedup vs baseline: 2.8763x; 1.0297x over previous
import functools

import jax
import jax.numpy as jnp
from jax import lax
from jax.experimental import pallas as pl
from jax.experimental.pallas import tpu as pltpu
from jax.experimental.pallas import tpu_sc as plsc

_ENT = 1000000
_REL = 1000
_D = 64
_B = 16384
_MARGIN = 1.0
_LS = 0.3
_LO = 0.3

_NW = 32
_PW = _B // _NW
_C = 64
_NCH = _PW // _C
_PK = 5 * _D


def _l2n_tc(x):
    return x * lax.rsqrt(jnp.maximum(jnp.sum(x * x, axis=-1, keepdims=True), 1e-12))


def _prep_body(rel_ref, wrs_ref, wro_ref, brs_ref, bro_ref, out_ref):
    rel = rel_ref[...]
    wrs = wrs_ref[...]
    wro = wro_ref[...]
    rn = _l2n_tc(rel)
    rh = _l2n_tc(brs_ref[...] * wrs)
    rt = _l2n_tc(bro_ref[...] * wro)
    out_ref[...] = jnp.concatenate([rn, wrs, wro, rh, rt], axis=1)


def _prep_rel(rel_emb, wrs_emb, wro_emb, brs_emb, bro_emb):
    return pl.pallas_call(
        _prep_body,
        out_shape=jax.ShapeDtypeStruct((_REL, _PK), jnp.float32),
    )(rel_emb, wrs_emb, wro_emb, brs_emb, bro_emb)


_BN = 8192
_NBLK = 62
_S = _NBLK * _BN


def _relay_body(top_ref, bot_ref, out_ref):
    a = top_ref[...]
    b = bot_ref[...]
    ia = lax.rsqrt(jnp.maximum(jnp.sum(a * a, axis=0, keepdims=True), 1e-12))
    ib = lax.rsqrt(jnp.maximum(jnp.sum(b * b, axis=0, keepdims=True), 1e-12))
    out_ref[:, 0:_D] = (a * ia).T
    out_ref[:, _D:2 * _D] = (b * ib).T


def _relayout(ent_t):
    return pl.pallas_call(
        _relay_body,
        grid=(_NBLK,),
        in_specs=[
            pl.BlockSpec((_D, _BN), lambda j: (0, j)),
            pl.BlockSpec((_D, _BN),
                         lambda j: (0, jnp.minimum(_NBLK + j, _ENT // _BN))),
        ],
        out_specs=pl.BlockSpec((_BN, 2 * _D), lambda j: (j, 0)),
        out_shape=jax.ShapeDtypeStruct((_S, 2 * _D), jnp.float32),
    )(ent_t, ent_t)


def _rsqrt16(x):
    x = jnp.maximum(x, 1e-12)
    i = lax.bitcast_convert_type(x, jnp.int32)
    i = jnp.int32(0x5F3759DF) - (i >> 1)
    y = lax.bitcast_convert_type(i, jnp.float32)
    for _ in range(3):
        y = y * (1.5 - 0.5 * x * y * y)
    return y


def _sc_kernel_fn(ent_hbm, rpk_hbm, ph_hbm, pt_hbm, pr_hbm, nh_hbm, nt_hbm, nr_hbm,
                  out_hbm,
                  ph_v, pt_v, pr_v, nh_v, nt_v, nr_v,
                  pht_v, ptt_v, nht_v, ntt_v,
                  hp_v, tp_v, hn_v, tn_v, rp_v, rn_v, acc_v, sem):
    info = plsc.get_sparse_core_info()
    nc = info.num_cores
    wid = lax.axis_index("s") * nc + lax.axis_index("c")
    base = wid * _PW

    pltpu.sync_copy(ph_hbm.at[pl.ds(base, _PW)], ph_v)
    pltpu.sync_copy(pt_hbm.at[pl.ds(base, _PW)], pt_v)
    pltpu.sync_copy(pr_hbm.at[pl.ds(base, _PW)], pr_v)
    pltpu.sync_copy(nh_hbm.at[pl.ds(base, _PW)], nh_v)
    pltpu.sync_copy(nt_hbm.at[pl.ds(base, _PW)], nt_v)
    pltpu.sync_copy(nr_hbm.at[pl.ds(base, _PW)], nr_v)

    def half(i, _):
        s = pl.ds(i * 16, 16)
        for idx_b, row_b in ((ph_v, pht_v), (pt_v, ptt_v),
                             (nh_v, nht_v), (nt_v, ntt_v)):
            v = idx_b[s]
            d = v - _S
            g = 1 - ((d >> 31) & 1)
            row_b[s] = v - g * _S
            idx_b[s] = g
        return 0
    lax.fori_loop(0, _PW // 16, half, 0)

    lanes = lax.iota(jnp.int32, 16)
    lane0 = lanes == 0

    dnums = lax.GatherDimensionNumbers(
        offset_dims=(), collapsed_slice_dims=(0,), start_index_map=(0,))

    def shuf(v, idx):
        return lax.gather(v, idx[:, None], dnums, (1,),
                          mode=lax.GatherScatterMode.PROMISE_IN_BOUNDS)

    def bsum(v):
        for sh in (1, 2, 4, 8):
            v = v + shuf(v, lanes ^ sh)
        return v

    def bcast_lane(v, k):
        return shuf(v, jnp.full((16,), k, jnp.int32))

    acc = jnp.zeros((16,), jnp.float32)
    for ch in range(_NCH):
        off = ch * _C
        d1 = pltpu.async_copy(ent_hbm.at[pht_v.at[pl.ds(off, _C)]], hp_v, sem)
        d2 = pltpu.async_copy(ent_hbm.at[ptt_v.at[pl.ds(off, _C)]], tp_v, sem)
        d3 = pltpu.async_copy(ent_hbm.at[nht_v.at[pl.ds(off, _C)]], hn_v, sem)
        d4 = pltpu.async_copy(ent_hbm.at[ntt_v.at[pl.ds(off, _C)]], tn_v, sem)
        d5 = pltpu.async_copy(rpk_hbm.at[pr_v.at[pl.ds(off, _C)]], rp_v, sem)
        d6 = pltpu.async_copy(rpk_hbm.at[nr_v.at[pl.ds(off, _C)]], rn_v, sem)
        d1.wait(); d2.wait(); d3.wait(); d4.wait(); d5.wait(); d6.wait()

        def body(e, acc):
            ebase = off + e
            al = ebase & -16
            lanev = jnp.full((16,), ebase & 15, jnp.int32)

            def parb(buf):
                return shuf(buf[pl.ds(al, 16)], lanev).astype(jnp.float32)

            php = parb(ph_v)
            ptp = parb(pt_v)
            nhp = parb(nh_v)
            ntp = parb(nt_v)

            def sel(buf, par, k):
                lo = buf[e, pl.ds(16 * k, 16)]
                hi = buf[e, pl.ds(_D + 16 * k, 16)]
                return lo * (1.0 - par) + hi * par

            hp = [sel(hp_v, php, k) for k in range(4)]
            tp = [sel(tp_v, ptp, k) for k in range(4)]
            hn = [sel(hn_v, nhp, k) for k in range(4)]
            tn = [sel(tn_v, ntp, k) for k in range(4)]
            wsp = [rp_v[e, pl.ds(_D + 16 * k, 16)] for k in range(4)]
            wop = [rp_v[e, pl.ds(2 * _D + 16 * k, 16)] for k in range(4)]
            wsn = [rn_v[e, pl.ds(_D + 16 * k, 16)] for k in range(4)]
            won = [rn_v[e, pl.ds(2 * _D + 16 * k, 16)] for k in range(4)]

            hwp = [hp[k] * wsp[k] for k in range(4)]
            twp = [tp[k] * wop[k] for k in range(4)]
            hwn = [hn[k] * wsn[k] for k in range(4)]
            twn = [tn[k] * won[k] for k in range(4)]

            def ssq(vs):
                a = vs[0] * vs[0]
                for k in range(1, 4):
                    a = a + vs[k] * vs[k]
                return bsum(a)

            sums = [ssq(hwp), ssq(twp), ssq(hwn), ssq(twn)]
            x = sums[0]
            for k in range(1, 4):
                x = jnp.where(lanes == k, sums[k], x)
            y = _rsqrt16(x)
            ihwp, itwp, ihwn, itwn = [bcast_lane(y, k) for k in range(4)]

            s1 = jnp.zeros((16,), jnp.float32)
            s2 = jnp.zeros((16,), jnp.float32)
            for k in range(4):
                rnp = rp_v[e, pl.ds(16 * k, 16)]
                rhp = rp_v[e, pl.ds(3 * _D + 16 * k, 16)]
                rtp = rp_v[e, pl.ds(4 * _D + 16 * k, 16)]
                rnn = rn_v[e, pl.ds(16 * k, 16)]
                rhn = rn_v[e, pl.ds(3 * _D + 16 * k, 16)]
                rtn = rn_v[e, pl.ds(4 * _D + 16 * k, 16)]
                ap = jnp.abs(hp[k] + rnp - tp[k])
                an = jnp.abs(hn[k] + rnn - tn[k])
                bp = jnp.abs(hwp[k] * ihwp - rhp) + jnp.abs(twp[k] * itwp - rtp)
                bn = jnp.abs(hwn[k] * ihwn - rhn) + jnp.abs(twn[k] * itwn - rtn)
                s1 = s1 + (ap - an)
                s2 = s2 + (bp - bn)
            dv = bsum(s1 + _LS * s2)
            hinge = jnp.maximum(dv + _MARGIN, 0.0)
            return acc + jnp.where(lane0, hinge, 0.0)

        acc = lax.fori_loop(0, _C, body, acc)

    acc_v[...] = acc
    pltpu.sync_copy(acc_v, out_hbm.at[wid])


def _sc_call(ent_emb, rel_packed, ph, pt, pr, nh, nt, nr):
    mesh = plsc.VectorSubcoreMesh(core_axis_name="c", subcore_axis_name="s")
    f = functools.partial(
        pl.kernel,
        mesh=mesh,
        compiler_params=pltpu.CompilerParams(use_tc_tiling_on_sc=False),
        out_type=jax.ShapeDtypeStruct((_NW, 16), jnp.float32),
        scratch_types=[
            pltpu.VMEM((_PW,), jnp.int32),
            pltpu.VMEM((_PW,), jnp.int32),
            pltpu.VMEM((_PW,), jnp.int32),
            pltpu.VMEM((_PW,), jnp.int32),
            pltpu.VMEM((_PW,), jnp.int32),
            pltpu.VMEM((_PW,), jnp.int32),
            pltpu.VMEM((_PW,), jnp.int32),
            pltpu.VMEM((_PW,), jnp.int32),
            pltpu.VMEM((_PW,), jnp.int32),
            pltpu.VMEM((_PW,), jnp.int32),
            pltpu.VMEM((_C, 2 * _D), jnp.float32),
            pltpu.VMEM((_C, 2 * _D), jnp.float32),
            pltpu.VMEM((_C, 2 * _D), jnp.float32),
            pltpu.VMEM((_C, 2 * _D), jnp.float32),
            pltpu.VMEM((_C, _PK), jnp.float32),
            pltpu.VMEM((_C, _PK), jnp.float32),
            pltpu.VMEM((16,), jnp.float32),
            pltpu.SemaphoreType.DMA,
        ],
    )(_sc_kernel_fn)
    return f(ent_emb, rel_packed, ph, pt, pr, nh, nt, nr)


def kernel(pos_h, pos_t, pos_r, neg_h, neg_t, neg_r,
           ent_emb, rel_emb, wrs_emb, wro_emb, brs_emb, bro_emb):
    rel_packed = _prep_rel(rel_emb, wrs_emb, wro_emb, brs_emb, bro_emb)
    ent_pairs = _relayout(ent_emb.T)
    parts = _sc_call(
        ent_pairs, rel_packed,
        pos_h.astype(jnp.int32), pos_t.astype(jnp.int32), pos_r.astype(jnp.int32),
        neg_h.astype(jnp.int32), neg_t.astype(jnp.int32), neg_r.astype(jnp.int32),
    )
    return jnp.sum(parts)


# speedup vs baseline: 3.0825x; 1.0717x over previous
import functools

import jax
import jax.numpy as jnp
from jax import lax
from jax.experimental import pallas as pl
from jax.experimental.pallas import tpu as pltpu
from jax.experimental.pallas import tpu_sc as plsc

_ENT = 1000000
_REL = 1000
_D = 64
_B = 16384
_MARGIN = 1.0
_LS = 0.3
_LO = 0.3

_NW = 32
_PW = _B // _NW
_C = 32
_NCH = _PW // _C
_PK = 5 * _D


def _l2n_tc(x):
    return x * lax.rsqrt(jnp.maximum(jnp.sum(x * x, axis=-1, keepdims=True), 1e-12))


def _prep_body(rel_ref, wrs_ref, wro_ref, brs_ref, bro_ref, out_ref):
    rel = rel_ref[...]
    wrs = wrs_ref[...]
    wro = wro_ref[...]
    rn = _l2n_tc(rel)
    rh = _l2n_tc(brs_ref[...] * wrs)
    rt = _l2n_tc(bro_ref[...] * wro)
    out_ref[...] = jnp.concatenate([rn, wrs, wro, rh, rt], axis=1)


def _prep_rel(rel_emb, wrs_emb, wro_emb, brs_emb, bro_emb):
    return pl.pallas_call(
        _prep_body,
        out_shape=jax.ShapeDtypeStruct((_REL, _PK), jnp.float32),
    )(rel_emb, wrs_emb, wro_emb, brs_emb, bro_emb)


_BN = 8192
_NBLK = 62
_S = _NBLK * _BN


def _relay_body(top_ref, bot_ref, out_ref):
    a = top_ref[...]
    b = bot_ref[...]
    ia = lax.rsqrt(jnp.maximum(jnp.sum(a * a, axis=0, keepdims=True), 1e-12))
    ib = lax.rsqrt(jnp.maximum(jnp.sum(b * b, axis=0, keepdims=True), 1e-12))
    out_ref[:, 0:_D] = (a * ia).T
    out_ref[:, _D:2 * _D] = (b * ib).T


def _relayout(ent_t):
    return pl.pallas_call(
        _relay_body,
        grid=(_NBLK,),
        in_specs=[
            pl.BlockSpec((_D, _BN), lambda j: (0, j)),
            pl.BlockSpec((_D, _BN),
                         lambda j: (0, jnp.minimum(_NBLK + j, _ENT // _BN))),
        ],
        out_specs=pl.BlockSpec((_BN, 2 * _D), lambda j: (j, 0)),
        out_shape=jax.ShapeDtypeStruct((_S, 2 * _D), jnp.float32),
    )(ent_t, ent_t)


def _rsqrt16(x):
    x = jnp.maximum(x, 1e-12)
    i = lax.bitcast_convert_type(x, jnp.int32)
    i = jnp.int32(0x5F3759DF) - (i >> 1)
    y = lax.bitcast_convert_type(i, jnp.float32)
    for _ in range(3):
        y = y * (1.5 - 0.5 * x * y * y)
    return y


def _sc_kernel_fn(ent_hbm, rpk_hbm, ph_hbm, pt_hbm, pr_hbm, nh_hbm, nt_hbm, nr_hbm,
                  out_hbm,
                  ph_v, pt_v, pr_v, nh_v, nt_v, nr_v,
                  pht_v, ptt_v, nht_v, ntt_v,
                  hpa_v, tpa_v, hna_v, tna_v, rpa_v, rna_v,
                  hpb_v, tpb_v, hnb_v, tnb_v, rpb_v, rnb_v,
                  acc_v, sema, semb):
    info = plsc.get_sparse_core_info()
    nc = info.num_cores
    wid = lax.axis_index("s") * nc + lax.axis_index("c")
    base = wid * _PW

    pltpu.sync_copy(ph_hbm.at[pl.ds(base, _PW)], ph_v)
    pltpu.sync_copy(pt_hbm.at[pl.ds(base, _PW)], pt_v)
    pltpu.sync_copy(pr_hbm.at[pl.ds(base, _PW)], pr_v)
    pltpu.sync_copy(nh_hbm.at[pl.ds(base, _PW)], nh_v)
    pltpu.sync_copy(nt_hbm.at[pl.ds(base, _PW)], nt_v)
    pltpu.sync_copy(nr_hbm.at[pl.ds(base, _PW)], nr_v)

    def half(i, _):
        s = pl.ds(i * 16, 16)
        for idx_b, row_b in ((ph_v, pht_v), (pt_v, ptt_v),
                             (nh_v, nht_v), (nt_v, ntt_v)):
            v = idx_b[s]
            d = v - _S
            g = 1 - ((d >> 31) & 1)
            row_b[s] = v - g * _S
            idx_b[s] = g
        return 0
    lax.fori_loop(0, _PW // 16, half, 0)

    lanes = lax.iota(jnp.int32, 16)
    lane0 = lanes == 0

    dnums = lax.GatherDimensionNumbers(
        offset_dims=(), collapsed_slice_dims=(0,), start_index_map=(0,))

    def shuf(v, idx):
        return lax.gather(v, idx[:, None], dnums, (1,),
                          mode=lax.GatherScatterMode.PROMISE_IN_BOUNDS)

    def bsum(v):
        for sh in (1, 2, 4, 8):
            v = v + shuf(v, lanes ^ sh)
        return v

    def bcast_lane(v, k):
        return shuf(v, jnp.full((16,), k, jnp.int32))

    def fire(ch, st):
        off = ch * _C
        hp_b, tp_b, hn_b, tn_b, rp_b, rn_b, sm = st
        return [
            pltpu.async_copy(ent_hbm.at[pht_v.at[pl.ds(off, _C)]], hp_b, sm),
            pltpu.async_copy(ent_hbm.at[ptt_v.at[pl.ds(off, _C)]], tp_b, sm),
            pltpu.async_copy(ent_hbm.at[nht_v.at[pl.ds(off, _C)]], hn_b, sm),
            pltpu.async_copy(ent_hbm.at[ntt_v.at[pl.ds(off, _C)]], tn_b, sm),
            pltpu.async_copy(rpk_hbm.at[pr_v.at[pl.ds(off, _C)]], rp_b, sm),
            pltpu.async_copy(rpk_hbm.at[nr_v.at[pl.ds(off, _C)]], rn_b, sm),
        ]

    def compute(acc, st, off):
        hp_v, tp_v, hn_v, tn_v, rp_v, rn_v, _ = st

        def body(e, acc):
            ebase = off + e
            al = ebase & -16
            lanev = jnp.full((16,), ebase & 15, jnp.int32)

            def parb(buf):
                return shuf(buf[pl.ds(al, 16)], lanev).astype(jnp.float32)

            php = parb(ph_v)
            ptp = parb(pt_v)
            nhp = parb(nh_v)
            ntp = parb(nt_v)

            def sel(buf, par, k):
                lo = buf[e, pl.ds(16 * k, 16)]
                hi = buf[e, pl.ds(_D + 16 * k, 16)]
                return lo * (1.0 - par) + hi * par

            hp = [sel(hp_v, php, k) for k in range(4)]
            tp = [sel(tp_v, ptp, k) for k in range(4)]
            hn = [sel(hn_v, nhp, k) for k in range(4)]
            tn = [sel(tn_v, ntp, k) for k in range(4)]
            wsp = [rp_v[e, pl.ds(_D + 16 * k, 16)] for k in range(4)]
            wop = [rp_v[e, pl.ds(2 * _D + 16 * k, 16)] for k in range(4)]
            wsn = [rn_v[e, pl.ds(_D + 16 * k, 16)] for k in range(4)]
            won = [rn_v[e, pl.ds(2 * _D + 16 * k, 16)] for k in range(4)]

            hwp = [hp[k] * wsp[k] for k in range(4)]
            twp = [tp[k] * wop[k] for k in range(4)]
            hwn = [hn[k] * wsn[k] for k in range(4)]
            twn = [tn[k] * won[k] for k in range(4)]

            def ssq(vs):
                a = vs[0] * vs[0]
                for k in range(1, 4):
                    a = a + vs[k] * vs[k]
                return bsum(a)

            sums = [ssq(hwp), ssq(twp), ssq(hwn), ssq(twn)]
            x = sums[0]
            for k in range(1, 4):
                x = jnp.where(lanes == k, sums[k], x)
            y = _rsqrt16(x)
            ihwp, itwp, ihwn, itwn = [bcast_lane(y, k) for k in range(4)]

            s1 = jnp.zeros((16,), jnp.float32)
            s2 = jnp.zeros((16,), jnp.float32)
            for k in range(4):
                rnp = rp_v[e, pl.ds(16 * k, 16)]
                rhp = rp_v[e, pl.ds(3 * _D + 16 * k, 16)]
                rtp = rp_v[e, pl.ds(4 * _D + 16 * k, 16)]
                rnn = rn_v[e, pl.ds(16 * k, 16)]
                rhn = rn_v[e, pl.ds(3 * _D + 16 * k, 16)]
                rtn = rn_v[e, pl.ds(4 * _D + 16 * k, 16)]
                ap = jnp.abs(hp[k] + rnp - tp[k])
                an = jnp.abs(hn[k] + rnn - tn[k])
                bp = jnp.abs(hwp[k] * ihwp - rhp) + jnp.abs(twp[k] * itwp - rtp)
                bn = jnp.abs(hwn[k] * ihwn - rhn) + jnp.abs(twn[k] * itwn - rtn)
                s1 = s1 + (ap - an)
                s2 = s2 + (bp - bn)
            dv = bsum(s1 + _LS * s2)
            hinge = jnp.maximum(dv + _MARGIN, 0.0)
            return acc + jnp.where(lane0, hinge, 0.0)

        return lax.fori_loop(0, _C, body, acc)

    sets = ((hpa_v, tpa_v, hna_v, tna_v, rpa_v, rna_v, sema),
            (hpb_v, tpb_v, hnb_v, tnb_v, rpb_v, rnb_v, semb))
    acc = jnp.zeros((16,), jnp.float32)
    handles = fire(0, sets[0])
    for ch in range(_NCH):
        par = ch & 1
        nxt = fire(ch + 1, sets[1 - par]) if ch + 1 < _NCH else None
        for h in handles:
            h.wait()
        acc = compute(acc, sets[par], ch * _C)
        handles = nxt

    acc_v[...] = acc
    pltpu.sync_copy(acc_v, out_hbm.at[wid])


def _sc_call(ent_emb, rel_packed, ph, pt, pr, nh, nt, nr):
    mesh = plsc.VectorSubcoreMesh(core_axis_name="c", subcore_axis_name="s")
    f = functools.partial(
        pl.kernel,
        mesh=mesh,
        compiler_params=pltpu.CompilerParams(use_tc_tiling_on_sc=False),
        out_type=jax.ShapeDtypeStruct((_NW, 16), jnp.float32),
        scratch_types=[
            pltpu.VMEM((_PW,), jnp.int32),
            pltpu.VMEM((_PW,), jnp.int32),
            pltpu.VMEM((_PW,), jnp.int32),
            pltpu.VMEM((_PW,), jnp.int32),
            pltpu.VMEM((_PW,), jnp.int32),
            pltpu.VMEM((_PW,), jnp.int32),
            pltpu.VMEM((_PW,), jnp.int32),
            pltpu.VMEM((_PW,), jnp.int32),
            pltpu.VMEM((_PW,), jnp.int32),
            pltpu.VMEM((_PW,), jnp.int32),
            pltpu.VMEM((_C, 2 * _D), jnp.float32),
            pltpu.VMEM((_C, 2 * _D), jnp.float32),
            pltpu.VMEM((_C, 2 * _D), jnp.float32),
            pltpu.VMEM((_C, 2 * _D), jnp.float32),
            pltpu.VMEM((_C, _PK), jnp.float32),
            pltpu.VMEM((_C, _PK), jnp.float32),
            pltpu.VMEM((_C, 2 * _D), jnp.float32),
            pltpu.VMEM((_C, 2 * _D), jnp.float32),
            pltpu.VMEM((_C, 2 * _D), jnp.float32),
            pltpu.VMEM((_C, 2 * _D), jnp.float32),
            pltpu.VMEM((_C, _PK), jnp.float32),
            pltpu.VMEM((_C, _PK), jnp.float32),
            pltpu.VMEM((16,), jnp.float32),
            pltpu.SemaphoreType.DMA,
            pltpu.SemaphoreType.DMA,
        ],
    )(_sc_kernel_fn)
    return f(ent_emb, rel_packed, ph, pt, pr, nh, nt, nr)


def kernel(pos_h, pos_t, pos_r, neg_h, neg_t, neg_r,
           ent_emb, rel_emb, wrs_emb, wro_emb, brs_emb, bro_emb):
    rel_packed = _prep_rel(rel_emb, wrs_emb, wro_emb, brs_emb, bro_emb)
    ent_pairs = _relayout(ent_emb.T)
    parts = _sc_call(
        ent_pairs, rel_packed,
        pos_h.astype(jnp.int32), pos_t.astype(jnp.int32), pos_r.astype(jnp.int32),
        neg_h.astype(jnp.int32), neg_t.astype(jnp.int32), neg_r.astype(jnp.int32),
    )
    return jnp.sum(parts)


# speedup vs baseline: 3.2479x; 1.0537x over previous
import functools

import jax
import jax.numpy as jnp
from jax import lax
from jax.experimental import pallas as pl
from jax.experimental.pallas import tpu as pltpu
from jax.experimental.pallas import tpu_sc as plsc

_ENT = 1000000
_REL = 1000
_D = 64
_B = 16384
_MARGIN = 1.0
_LS = 0.3
_LO = 0.3

_NW = 32
_PW = _B // _NW
_C = 32
_NCH = _PW // _C
_PK = 5 * _D


def _l2n_tc(x):
    return x * lax.rsqrt(jnp.maximum(jnp.sum(x * x, axis=-1, keepdims=True), 1e-12))


def _prep_body(rel_ref, wrs_ref, wro_ref, brs_ref, bro_ref, out_ref):
    rel = rel_ref[...]
    wrs = wrs_ref[...]
    wro = wro_ref[...]
    rn = _l2n_tc(rel)
    rh = _l2n_tc(brs_ref[...] * wrs)
    rt = _l2n_tc(bro_ref[...] * wro)
    out_ref[...] = jnp.concatenate([rn, wrs, wro, rh, rt], axis=1)


def _prep_rel(rel_emb, wrs_emb, wro_emb, brs_emb, bro_emb):
    return pl.pallas_call(
        _prep_body,
        out_shape=jax.ShapeDtypeStruct((_REL, _PK), jnp.float32),
    )(rel_emb, wrs_emb, wro_emb, brs_emb, bro_emb)


_BN = 16384
_NBLK = 31
_S = _NBLK * _BN


def _relay_body(top_ref, bot_ref, out_ref):
    a = top_ref[...]
    b = bot_ref[...]
    ia = lax.rsqrt(jnp.maximum(jnp.sum(a * a, axis=0, keepdims=True), 1e-12))
    ib = lax.rsqrt(jnp.maximum(jnp.sum(b * b, axis=0, keepdims=True), 1e-12))
    out_ref[:, 0:_D] = (a * ia).T
    out_ref[:, _D:2 * _D] = (b * ib).T


def _relayout(ent_t):
    return pl.pallas_call(
        _relay_body,
        grid=(_NBLK,),
        in_specs=[
            pl.BlockSpec((_D, _BN), lambda j: (0, j)),
            pl.BlockSpec((_D, _BN),
                         lambda j: (0, jnp.minimum(_NBLK + j, _ENT // _BN))),
        ],
        out_specs=pl.BlockSpec((_BN, 2 * _D), lambda j: (j, 0)),
        out_shape=jax.ShapeDtypeStruct((_S, 2 * _D), jnp.float32),
    )(ent_t, ent_t)


def _rsqrt16(x):
    x = jnp.maximum(x, 1e-12)
    i = lax.bitcast_convert_type(x, jnp.int32)
    i = jnp.int32(0x5F3759DF) - (i >> 1)
    y = lax.bitcast_convert_type(i, jnp.float32)
    for _ in range(3):
        y = y * (1.5 - 0.5 * x * y * y)
    return y


def _sc_kernel_fn(ent_hbm, rpk_hbm, ph_hbm, pt_hbm, pr_hbm, nh_hbm, nt_hbm, nr_hbm,
                  out_hbm,
                  ph_v, pt_v, pr_v, nh_v, nt_v, nr_v,
                  pht_v, ptt_v, nht_v, ntt_v,
                  hpa_v, tpa_v, hna_v, tna_v, rpa_v, rna_v,
                  hpb_v, tpb_v, hnb_v, tnb_v, rpb_v, rnb_v,
                  acc_v, sema, semb):
    info = plsc.get_sparse_core_info()
    nc = info.num_cores
    wid = lax.axis_index("s") * nc + lax.axis_index("c")
    base = wid * _PW

    pltpu.sync_copy(ph_hbm.at[pl.ds(base, _PW)], ph_v)
    pltpu.sync_copy(pt_hbm.at[pl.ds(base, _PW)], pt_v)
    pltpu.sync_copy(pr_hbm.at[pl.ds(base, _PW)], pr_v)
    pltpu.sync_copy(nh_hbm.at[pl.ds(base, _PW)], nh_v)
    pltpu.sync_copy(nt_hbm.at[pl.ds(base, _PW)], nt_v)
    pltpu.sync_copy(nr_hbm.at[pl.ds(base, _PW)], nr_v)

    def half(i, _):
        s = pl.ds(i * 16, 16)
        for idx_b, row_b in ((ph_v, pht_v), (pt_v, ptt_v),
                             (nh_v, nht_v), (nt_v, ntt_v)):
            v = idx_b[s]
            d = v - _S
            g = 1 - ((d >> 31) & 1)
            row_b[s] = v - g * _S
            idx_b[s] = g
        return 0
    lax.fori_loop(0, _PW // 16, half, 0)

    lanes = lax.iota(jnp.int32, 16)
    lane0 = lanes == 0

    dnums = lax.GatherDimensionNumbers(
        offset_dims=(), collapsed_slice_dims=(0,), start_index_map=(0,))

    def shuf(v, idx):
        return lax.gather(v, idx[:, None], dnums, (1,),
                          mode=lax.GatherScatterMode.PROMISE_IN_BOUNDS)

    def bsum(v):
        for sh in (1, 2, 4, 8):
            v = v + shuf(v, lanes ^ sh)
        return v

    def bcast_lane(v, k):
        return shuf(v, jnp.full((16,), k, jnp.int32))

    def fire(ch, st):
        off = ch * _C
        hp_b, tp_b, hn_b, tn_b, rp_b, rn_b, sm = st
        return [
            pltpu.async_copy(ent_hbm.at[pht_v.at[pl.ds(off, _C)]], hp_b, sm),
            pltpu.async_copy(ent_hbm.at[ptt_v.at[pl.ds(off, _C)]], tp_b, sm),
            pltpu.async_copy(ent_hbm.at[nht_v.at[pl.ds(off, _C)]], hn_b, sm),
            pltpu.async_copy(ent_hbm.at[ntt_v.at[pl.ds(off, _C)]], tn_b, sm),
            pltpu.async_copy(rpk_hbm.at[pr_v.at[pl.ds(off, _C)]], rp_b, sm),
            pltpu.async_copy(rpk_hbm.at[nr_v.at[pl.ds(off, _C)]], rn_b, sm),
        ]

    def compute(acc, st, off):
        hp_v, tp_v, hn_v, tn_v, rp_v, rn_v, _ = st

        def body(e, acc):
            ebase = off + e
            al = ebase & -16
            lanev = jnp.full((16,), ebase & 15, jnp.int32)

            def parb(buf):
                return shuf(buf[pl.ds(al, 16)], lanev).astype(jnp.float32)

            php = parb(ph_v)
            ptp = parb(pt_v)
            nhp = parb(nh_v)
            ntp = parb(nt_v)

            def sel(buf, par, k):
                lo = buf[e, pl.ds(16 * k, 16)]
                hi = buf[e, pl.ds(_D + 16 * k, 16)]
                return lo * (1.0 - par) + hi * par

            hp = [sel(hp_v, php, k) for k in range(4)]
            tp = [sel(tp_v, ptp, k) for k in range(4)]
            hn = [sel(hn_v, nhp, k) for k in range(4)]
            tn = [sel(tn_v, ntp, k) for k in range(4)]
            wsp = [rp_v[e, pl.ds(_D + 16 * k, 16)] for k in range(4)]
            wop = [rp_v[e, pl.ds(2 * _D + 16 * k, 16)] for k in range(4)]
            wsn = [rn_v[e, pl.ds(_D + 16 * k, 16)] for k in range(4)]
            won = [rn_v[e, pl.ds(2 * _D + 16 * k, 16)] for k in range(4)]

            hwp = [hp[k] * wsp[k] for k in range(4)]
            twp = [tp[k] * wop[k] for k in range(4)]
            hwn = [hn[k] * wsn[k] for k in range(4)]
            twn = [tn[k] * won[k] for k in range(4)]

            def ssq(vs):
                a = vs[0] * vs[0]
                for k in range(1, 4):
                    a = a + vs[k] * vs[k]
                return bsum(a)

            sums = [ssq(hwp), ssq(twp), ssq(hwn), ssq(twn)]
            x = sums[0]
            for k in range(1, 4):
                x = jnp.where(lanes == k, sums[k], x)
            y = _rsqrt16(x)
            ihwp, itwp, ihwn, itwn = [bcast_lane(y, k) for k in range(4)]

            s1 = jnp.zeros((16,), jnp.float32)
            s2 = jnp.zeros((16,), jnp.float32)
            for k in range(4):
                rnp = rp_v[e, pl.ds(16 * k, 16)]
                rhp = rp_v[e, pl.ds(3 * _D + 16 * k, 16)]
                rtp = rp_v[e, pl.ds(4 * _D + 16 * k, 16)]
                rnn = rn_v[e, pl.ds(16 * k, 16)]
                rhn = rn_v[e, pl.ds(3 * _D + 16 * k, 16)]
                rtn = rn_v[e, pl.ds(4 * _D + 16 * k, 16)]
                ap = jnp.abs(hp[k] + rnp - tp[k])
                an = jnp.abs(hn[k] + rnn - tn[k])
                bp = jnp.abs(hwp[k] * ihwp - rhp) + jnp.abs(twp[k] * itwp - rtp)
                bn = jnp.abs(hwn[k] * ihwn - rhn) + jnp.abs(twn[k] * itwn - rtn)
                s1 = s1 + (ap - an)
                s2 = s2 + (bp - bn)
            dv = bsum(s1 + _LS * s2)
            hinge = jnp.maximum(dv + _MARGIN, 0.0)
            return acc + jnp.where(lane0, hinge, 0.0)

        return lax.fori_loop(0, _C, body, acc)

    sets = ((hpa_v, tpa_v, hna_v, tna_v, rpa_v, rna_v, sema),
            (hpb_v, tpb_v, hnb_v, tnb_v, rpb_v, rnb_v, semb))
    acc = jnp.zeros((16,), jnp.float32)
    handles = fire(0, sets[0])
    for ch in range(_NCH):
        par = ch & 1
        nxt = fire(ch + 1, sets[1 - par]) if ch + 1 < _NCH else None
        for h in handles:
            h.wait()
        acc = compute(acc, sets[par], ch * _C)
        handles = nxt

    acc_v[...] = acc
    pltpu.sync_copy(acc_v, out_hbm.at[wid])


def _sc_call(ent_emb, rel_packed, ph, pt, pr, nh, nt, nr):
    mesh = plsc.VectorSubcoreMesh(core_axis_name="c", subcore_axis_name="s")
    f = functools.partial(
        pl.kernel,
        mesh=mesh,
        compiler_params=pltpu.CompilerParams(use_tc_tiling_on_sc=False),
        out_type=jax.ShapeDtypeStruct((_NW, 16), jnp.float32),
        scratch_types=[
            pltpu.VMEM((_PW,), jnp.int32),
            pltpu.VMEM((_PW,), jnp.int32),
            pltpu.VMEM((_PW,), jnp.int32),
            pltpu.VMEM((_PW,), jnp.int32),
            pltpu.VMEM((_PW,), jnp.int32),
            pltpu.VMEM((_PW,), jnp.int32),
            pltpu.VMEM((_PW,), jnp.int32),
            pltpu.VMEM((_PW,), jnp.int32),
            pltpu.VMEM((_PW,), jnp.int32),
            pltpu.VMEM((_PW,), jnp.int32),
            pltpu.VMEM((_C, 2 * _D), jnp.float32),
            pltpu.VMEM((_C, 2 * _D), jnp.float32),
            pltpu.VMEM((_C, 2 * _D), jnp.float32),
            pltpu.VMEM((_C, 2 * _D), jnp.float32),
            pltpu.VMEM((_C, _PK), jnp.float32),
            pltpu.VMEM((_C, _PK), jnp.float32),
            pltpu.VMEM((_C, 2 * _D), jnp.float32),
            pltpu.VMEM((_C, 2 * _D), jnp.float32),
            pltpu.VMEM((_C, 2 * _D), jnp.float32),
            pltpu.VMEM((_C, 2 * _D), jnp.float32),
            pltpu.VMEM((_C, _PK), jnp.float32),
            pltpu.VMEM((_C, _PK), jnp.float32),
            pltpu.VMEM((16,), jnp.float32),
            pltpu.SemaphoreType.DMA,
            pltpu.SemaphoreType.DMA,
        ],
    )(_sc_kernel_fn)
    return f(ent_emb, rel_packed, ph, pt, pr, nh, nt, nr)


def kernel(pos_h, pos_t, pos_r, neg_h, neg_t, neg_r,
           ent_emb, rel_emb, wrs_emb, wro_emb, brs_emb, bro_emb):
    rel_packed = _prep_rel(rel_emb, wrs_emb, wro_emb, brs_emb, bro_emb)
    ent_pairs = _relayout(ent_emb.T)
    parts = _sc_call(
        ent_pairs, rel_packed,
        pos_h.astype(jnp.int32), pos_t.astype(jnp.int32), pos_r.astype(jnp.int32),
        neg_h.astype(jnp.int32), neg_t.astype(jnp.int32), neg_r.astype(jnp.int32),
    )
    return jnp.sum(parts)
